# Initial kernel scaffold; baseline (speedup 1.0000x reference)
import jax
import jax.numpy as jnp
from jax.experimental import pallas as pl


def kernel(x, W_gate, Wg, Wu, Wd):
    raise NotImplementedError("write your pallas kernel here")


# speedup vs baseline: 1.3948x; 1.3948x over previous
import functools

import jax
import jax.numpy as jnp
from jax import lax
from jax.experimental import pallas as pl
from jax.experimental.pallas import tpu as pltpu

HIDDEN = 1024
INTER = 2048
NUM_EXPERTS = 8
TOP_K = 2
AUX_COEF = 0.001

TOK_BLK = 1024
INT_BLK = 512
NT = 4096 // TOK_BLK
NI = INTER // INT_BLK


def _router_block(xb, wg_all):
    logits = jax.lax.dot_general(
        xb, wg_all, (((1,), (1,)), ((), ())),
        preferred_element_type=jnp.float32)
    m = jnp.max(logits, axis=1, keepdims=True)
    ex = jnp.exp(logits - m)
    probs = ex / jnp.sum(ex, axis=1, keepdims=True)

    eidx = lax.broadcasted_iota(jnp.int32, probs.shape, 1)
    p0 = jnp.max(probs, axis=1, keepdims=True)
    is0 = (probs == p0)
    e0 = jnp.min(jnp.where(is0, eidx, NUM_EXPERTS), axis=1, keepdims=True)
    oh0 = (eidx == e0).astype(jnp.float32)
    masked = jnp.where(eidx == e0, -jnp.inf, probs)
    p1 = jnp.max(masked, axis=1, keepdims=True)
    is1 = (masked == p1)
    e1 = jnp.min(jnp.where(is1, eidx, NUM_EXPERTS), axis=1, keepdims=True)
    oh1 = (eidx == e1).astype(jnp.float32)

    denom = p0 + p1
    comb = (oh0 * (p0 / denom)) + (oh1 * (p1 / denom))
    sel = oh0 + oh1
    return comb, probs, sel


def _moe_dense_kernel(x_ref, wgate_ref, wg_ref, wu_ref, wd_ref,
                      y_ref, aux_ref, comb_ref, psum_ref, cnt_ref):
    t = pl.program_id(0)
    e = pl.program_id(1)
    i = pl.program_id(2)

    xb = x_ref[...]

    @pl.when(jnp.logical_and(e == 0, i == 0))
    def _router():
        comb, probs, sel = _router_block(xb, wgate_ref[...])
        comb_ref[...] = comb

        @pl.when(t == 0)
        def _init():
            psum_ref[...] = jnp.zeros_like(psum_ref)
            cnt_ref[...] = jnp.zeros_like(cnt_ref)

        psum_ref[...] += jnp.sum(probs * sel, axis=0, keepdims=True)
        cnt_ref[...] += jnp.sum(sel, axis=0, keepdims=True)

    @pl.when(jnp.logical_and(e == 0, i == 0))
    def _init_y():
        y_ref[...] = jnp.zeros_like(y_ref)

    wg_t = wg_ref[0]
    wu_t = wu_ref[0]
    wd_t = wd_ref[0]
    g = jax.lax.dot_general(xb, wg_t, (((1,), (1,)), ((), ())),
                            preferred_element_type=jnp.float32)
    u = jax.lax.dot_general(xb, wu_t, (((1,), (1,)), ((), ())),
                            preferred_element_type=jnp.float32)
    h = (g * jax.nn.sigmoid(g)) * u
    part = jax.lax.dot_general(h, wd_t, (((1,), (1,)), ((), ())),
                               preferred_element_type=jnp.float32)

    eidx = lax.broadcasted_iota(jnp.int32, comb_ref.shape, 1)
    wcol = jnp.sum(comb_ref[...] * (eidx == e).astype(jnp.float32),
                   axis=1, keepdims=True)
    y_ref[...] += part * wcol

    @pl.when(jnp.logical_and(t == NT - 1,
                             jnp.logical_and(e == NUM_EXPERTS - 1, i == NI - 1)))
    def _aux():
        n_tok = jnp.float32(NT * TOK_BLK)
        p_expert = psum_ref[...] / n_tok
        p_tok = cnt_ref[...] / (n_tok * TOP_K)
        aux_ref[0, 0] = jnp.sum(p_expert * p_tok) * NUM_EXPERTS * AUX_COEF


@functools.partial(jax.jit, static_argnames=("interpret",))
def _moe_dense(x2d, W_gate, Wg, Wu, Wd, interpret=False):
    y, aux = pl.pallas_call(
        _moe_dense_kernel,
        grid=(NT, NUM_EXPERTS, NI),
        in_specs=[
            pl.BlockSpec((TOK_BLK, HIDDEN), lambda t, e, i: (t, 0)),
            pl.BlockSpec((NUM_EXPERTS, HIDDEN), lambda t, e, i: (0, 0)),
            pl.BlockSpec((1, INT_BLK, HIDDEN), lambda t, e, i: (e, i, 0)),
            pl.BlockSpec((1, INT_BLK, HIDDEN), lambda t, e, i: (e, i, 0)),
            pl.BlockSpec((1, HIDDEN, INT_BLK), lambda t, e, i: (e, 0, i)),
        ],
        out_specs=[
            pl.BlockSpec((TOK_BLK, HIDDEN), lambda t, e, i: (t, 0)),
            pl.BlockSpec(memory_space=pltpu.SMEM),
        ],
        out_shape=[
            jax.ShapeDtypeStruct((4096, HIDDEN), jnp.float32),
            jax.ShapeDtypeStruct((1, 1), jnp.float32),
        ],
        scratch_shapes=[
            pltpu.VMEM((TOK_BLK, NUM_EXPERTS), jnp.float32),
            pltpu.VMEM((1, NUM_EXPERTS), jnp.float32),
            pltpu.VMEM((1, NUM_EXPERTS), jnp.float32),
        ],
        interpret=interpret,
    )(x2d, W_gate, Wg, Wu, Wd)
    return y, aux[0, 0]


def kernel(x, W_gate, Wg, Wu, Wd):
    bsz, seq, hid = x.shape
    x2d = x.reshape(-1, hid)
    y, aux = _moe_dense(x2d, W_gate, Wg, Wu, Wd)
    return y.reshape(bsz, seq, hid), aux


# speedup vs baseline: 1.6761x; 1.2017x over previous
import functools

import jax
import jax.numpy as jnp
from jax import lax
from jax.experimental import pallas as pl
from jax.experimental.pallas import tpu as pltpu
from jax.experimental.pallas import tpu_sc as plsc

HIDDEN = 1024
INTER = 2048
NUM_EXPERTS = 8
TOP_K = 2
AUX_COEF = 0.001
T = 4096
ASSIGN = T * TOP_K

BM = 512
ROWS = ASSIGN + NUM_EXPERTS * BM
NB = ROWS // BM
INT_BLK = 512
NI = INTER // INT_BLK

RCHUNK = 128
NRC = T // RCHUNK

NW = 32
TPW = T // NW
DCHUNK = 64
CCHUNK = 32


def _router_kernel(x_ref, wgate_ref,
                   pos0_ref, pos1_ref, w0_ref, w1_ref, bexp_ref, aux_ref,
                   e0_ref, e1_ref):
    wgate = wgate_ref[...]
    slt = (lax.broadcasted_iota(jnp.int32, (RCHUNK, RCHUNK), 0)
           > lax.broadcasted_iota(jnp.int32, (RCHUNK, RCHUNK), 1)
           ).astype(jnp.float32)

    running = jnp.zeros((1, NUM_EXPERTS), jnp.float32)
    psum = jnp.zeros((1, NUM_EXPERTS), jnp.float32)
    for c in range(NRC):
        rows = pl.ds(c * RCHUNK, RCHUNK)
        xb = x_ref[rows, :]
        logits = lax.dot_general(xb, wgate, (((1,), (1,)), ((), ())),
                                 preferred_element_type=jnp.float32)
        m = jnp.max(logits, axis=1, keepdims=True)
        ex = jnp.exp(logits - m)
        probs = ex / jnp.sum(ex, axis=1, keepdims=True)

        eidx = lax.broadcasted_iota(jnp.int32, probs.shape, 1)
        p0 = jnp.max(probs, axis=1, keepdims=True)
        e0 = jnp.min(jnp.where(probs == p0, eidx, NUM_EXPERTS),
                     axis=1, keepdims=True)
        oh0 = (eidx == e0).astype(jnp.float32)
        masked = jnp.where(eidx == e0, -jnp.inf, probs)
        p1 = jnp.max(masked, axis=1, keepdims=True)
        e1 = jnp.min(jnp.where(masked == p1, eidx, NUM_EXPERTS),
                     axis=1, keepdims=True)
        oh1 = (eidx == e1).astype(jnp.float32)

        denom = p0 + p1
        w0_ref[rows, :] = p0 / denom
        w1_ref[rows, :] = p1 / denom
        e0_ref[rows, :] = e0
        e1_ref[rows, :] = e1

        h = oh0 + oh1
        intra = lax.dot_general(slt, h, (((1,), (0,)), ((), ())),
                                preferred_element_type=jnp.float32)
        cums = intra + running
        pos0_ref[rows, :] = jnp.sum(cums * oh0, axis=1,
                                    keepdims=True).astype(jnp.int32)
        pos1_ref[rows, :] = jnp.sum(cums * oh1, axis=1,
                                    keepdims=True).astype(jnp.int32)
        running = running + jnp.sum(h, axis=0, keepdims=True)
        psum = psum + jnp.sum(probs * h, axis=0, keepdims=True)

    counts = running
    cnt_i = counts.astype(jnp.int32)
    pc = ((cnt_i + (BM - 1)) // BM) * BM
    sut = (lax.broadcasted_iota(jnp.int32, (NUM_EXPERTS, NUM_EXPERTS), 0)
           < lax.broadcasted_iota(jnp.int32, (NUM_EXPERTS, NUM_EXPERTS), 1)
           ).astype(jnp.float32)
    starts = lax.dot_general(pc.astype(jnp.float32), sut,
                             (((1,), (0,)), ((), ())),
                             preferred_element_type=jnp.float32
                             ).astype(jnp.int32)

    for c in range(NRC):
        rows = pl.ds(c * RCHUNK, RCHUNK)
        eidx = lax.broadcasted_iota(jnp.int32, (RCHUNK, NUM_EXPERTS), 1)
        oh0 = (eidx == e0_ref[rows, :]).astype(jnp.int32)
        oh1 = (eidx == e1_ref[rows, :]).astype(jnp.int32)
        pos0_ref[rows, :] += jnp.sum(starts * oh0, axis=1, keepdims=True)
        pos1_ref[rows, :] += jnp.sum(starts * oh1, axis=1, keepdims=True)

    bs = lax.broadcasted_iota(jnp.int32, (NB, NUM_EXPERTS), 0) * BM
    eix = lax.broadcasted_iota(jnp.int32, (NB, NUM_EXPERTS), 1)
    hit = jnp.logical_and(bs >= starts, bs < starts + pc).astype(jnp.int32)
    bexp_ref[...] = jnp.sum(eix * hit, axis=1, keepdims=True)

    p_expert = psum / jnp.float32(T)
    p_tok = counts / jnp.float32(ASSIGN)
    aux_ref[0, 0] = jnp.sum(p_expert * p_tok) * NUM_EXPERTS * AUX_COEF


def _router(x2d, W_gate, interpret=False):
    outs = pl.pallas_call(
        _router_kernel,
        in_specs=[
            pl.BlockSpec((T, HIDDEN), lambda: (0, 0)),
            pl.BlockSpec((NUM_EXPERTS, HIDDEN), lambda: (0, 0)),
        ],
        out_specs=[
            pl.BlockSpec((T, 1), lambda: (0, 0)),
            pl.BlockSpec((T, 1), lambda: (0, 0)),
            pl.BlockSpec((T, 1), lambda: (0, 0)),
            pl.BlockSpec((T, 1), lambda: (0, 0)),
            pl.BlockSpec((NB, 1), lambda: (0, 0)),
            pl.BlockSpec(memory_space=pltpu.SMEM),
        ],
        out_shape=[
            jax.ShapeDtypeStruct((T, 1), jnp.int32),
            jax.ShapeDtypeStruct((T, 1), jnp.int32),
            jax.ShapeDtypeStruct((T, 1), jnp.float32),
            jax.ShapeDtypeStruct((T, 1), jnp.float32),
            jax.ShapeDtypeStruct((NB, 1), jnp.int32),
            jax.ShapeDtypeStruct((1, 1), jnp.float32),
        ],
        scratch_shapes=[
            pltpu.VMEM((T, 1), jnp.int32),
            pltpu.VMEM((T, 1), jnp.int32),
        ],
        interpret=interpret,
    )(x2d, W_gate)
    return outs


def _sc_mesh():
    return plsc.VectorSubcoreMesh(core_axis_name="c", subcore_axis_name="s",
                                  num_cores=2, num_subcores=16)


def _dispatch_body(x_hbm, pos0_hbm, pos1_hbm, xs_hbm,
                   idx0_v, idx1_v, rows_v, sem0, sem1):
    wid = lax.axis_index("s") * 2 + lax.axis_index("c")
    for sub in range(TPW // DCHUNK):
        base = wid * TPW + sub * DCHUNK
        pltpu.sync_copy(pos0_hbm.at[pl.ds(base, DCHUNK)], idx0_v)
        pltpu.sync_copy(pos1_hbm.at[pl.ds(base, DCHUNK)], idx1_v)
        pltpu.sync_copy(x_hbm.at[pl.ds(base, DCHUNK)], rows_v)
        d0 = pltpu.async_copy(rows_v, xs_hbm.at[idx0_v], sem0)
        d1 = pltpu.async_copy(rows_v, xs_hbm.at[idx1_v], sem1)
        d0.wait()
        d1.wait()


@functools.lru_cache(maxsize=None)
def _make_dispatch():
    return pl.kernel(
        _dispatch_body,
        out_type=jax.ShapeDtypeStruct((ROWS, HIDDEN), jnp.float32),
        mesh=_sc_mesh(),
        scratch_types=[
            pltpu.VMEM((DCHUNK,), jnp.int32),
            pltpu.VMEM((DCHUNK,), jnp.int32),
            pltpu.VMEM((DCHUNK, HIDDEN), jnp.float32),
            pltpu.SemaphoreType.DMA,
            pltpu.SemaphoreType.DMA,
        ],
    )


def _ffn_kernel(bexp_ref, xs_ref, wg_ref, wu_ref, wd_ref, ys_ref):
    i = pl.program_id(1)

    @pl.when(i == 0)
    def _init():
        ys_ref[...] = jnp.zeros_like(ys_ref)

    xb = xs_ref[...]
    g = lax.dot_general(xb, wg_ref[0], (((1,), (1,)), ((), ())),
                        preferred_element_type=jnp.float32)
    u = lax.dot_general(xb, wu_ref[0], (((1,), (1,)), ((), ())),
                        preferred_element_type=jnp.float32)
    h = (g * jax.nn.sigmoid(g)) * u
    ys_ref[...] += lax.dot_general(h, wd_ref[0], (((1,), (1,)), ((), ())),
                                   preferred_element_type=jnp.float32)


def _ffn(bexp, xs, Wg, Wu, Wd, interpret=False):
    grid_spec = pltpu.PrefetchScalarGridSpec(
        num_scalar_prefetch=1,
        grid=(NB, NI),
        in_specs=[
            pl.BlockSpec((BM, HIDDEN), lambda b, i, be: (b, 0)),
            pl.BlockSpec((1, INT_BLK, HIDDEN), lambda b, i, be: (be[b], i, 0)),
            pl.BlockSpec((1, INT_BLK, HIDDEN), lambda b, i, be: (be[b], i, 0)),
            pl.BlockSpec((1, HIDDEN, INT_BLK), lambda b, i, be: (be[b], 0, i)),
        ],
        out_specs=pl.BlockSpec((BM, HIDDEN), lambda b, i, be: (b, 0)),
    )
    return pl.pallas_call(
        _ffn_kernel,
        grid_spec=grid_spec,
        out_shape=jax.ShapeDtypeStruct((ROWS, HIDDEN), jnp.float32),
        interpret=interpret,
    )(bexp, xs, Wg, Wu, Wd)


def _combine_body(ys_hbm, pos0_hbm, pos1_hbm, w0_hbm, w1_hbm, out_hbm,
                  idx0_v, idx1_v, w0_v, w1_v, a_v, b_v, sem0, sem1):
    wid = lax.axis_index("s") * 2 + lax.axis_index("c")
    for sub in range(TPW // CCHUNK):
        base = wid * TPW + sub * CCHUNK
        pltpu.sync_copy(pos0_hbm.at[pl.ds(base, CCHUNK)], idx0_v)
        pltpu.sync_copy(pos1_hbm.at[pl.ds(base, CCHUNK)], idx1_v)
        pltpu.sync_copy(w0_hbm.at[pl.ds(base, CCHUNK)], w0_v)
        pltpu.sync_copy(w1_hbm.at[pl.ds(base, CCHUNK)], w1_v)
        d0 = pltpu.async_copy(ys_hbm.at[idx0_v], a_v, sem0)
        d1 = pltpu.async_copy(ys_hbm.at[idx1_v], b_v, sem1)
        d0.wait()
        d1.wait()
        wvecs = [(w0_v[pl.ds(g * 16, 16)], w1_v[pl.ds(g * 16, 16)])
                 for g in range(CCHUNK // 16)]
        for i in range(CCHUNK):
            w0s = wvecs[i // 16][0][i % 16]
            w1s = wvecs[i // 16][1][i % 16]

            def body(j, _, i=i, w0s=w0s, w1s=w1s):
                sl = pl.ds(j * 16, 16)
                a_v[i, sl] = a_v[i, sl] * w0s + b_v[i, sl] * w1s
                return 0

            lax.fori_loop(0, HIDDEN // 16, body, 0)
        pltpu.sync_copy(a_v, out_hbm.at[pl.ds(base, CCHUNK)])


@functools.lru_cache(maxsize=None)
def _make_combine():
    return pl.kernel(
        _combine_body,
        out_type=jax.ShapeDtypeStruct((T, HIDDEN), jnp.float32),
        mesh=_sc_mesh(),
        scratch_types=[
            pltpu.VMEM((CCHUNK,), jnp.int32),
            pltpu.VMEM((CCHUNK,), jnp.int32),
            pltpu.VMEM((CCHUNK,), jnp.float32),
            pltpu.VMEM((CCHUNK,), jnp.float32),
            pltpu.VMEM((CCHUNK, HIDDEN), jnp.float32),
            pltpu.VMEM((CCHUNK, HIDDEN), jnp.float32),
            pltpu.SemaphoreType.DMA,
            pltpu.SemaphoreType.DMA,
        ],
    )


@jax.jit
def _moe(x2d, W_gate, Wg, Wu, Wd):
    pos0, pos1, w0, w1, bexp, aux = _router(x2d, W_gate)
    p0 = pos0.reshape(T)
    p1 = pos1.reshape(T)
    xs = _make_dispatch()(x2d, p0, p1)
    ys = _ffn(bexp.reshape(NB), xs, Wg, Wu, Wd)
    y2d = _make_combine()(ys, p0, p1, w0.reshape(T), w1.reshape(T))
    return y2d, aux[0, 0]


def kernel(x, W_gate, Wg, Wu, Wd):
    bsz, seq, hid = x.shape
    x2d = x.reshape(-1, hid)
    y, aux = _moe(x2d, W_gate, Wg, Wu, Wd)
    return y.reshape(bsz, seq, hid), aux


# speedup vs baseline: 2.1281x; 1.2696x over previous
import functools

import jax
import jax.numpy as jnp
from jax import lax
from jax.experimental import pallas as pl
from jax.experimental.pallas import tpu as pltpu
from jax.experimental.pallas import tpu_sc as plsc

HIDDEN = 1024
INTER = 2048
NUM_EXPERTS = 8
TOP_K = 2
AUX_COEF = 0.001
T = 4096
ASSIGN = T * TOP_K

BM = 512
ROWS = ASSIGN + NUM_EXPERTS * BM
NB = ROWS // BM
NB1 = NB + 1
INT_BLK = 512
NI = INTER // INT_BLK

RCH = 1024
SCH = 128

NW = 32
TPW = T // NW
DCHUNK = 64
CCHUNK = 32


def _router_kernel(x_ref, wgate_ref,
                   pos0_ref, pos1_ref, w0_ref, w1_ref, bexp_ref, aux_ref,
                   e0_ref, e1_ref, h_ref, s_ref):
    wgate = wgate_ref[...]
    psum = jnp.zeros((1, NUM_EXPERTS), jnp.float32)

    g8 = (lax.broadcasted_iota(jnp.int32, (RCH // SCH, RCH), 1) // SCH
          == lax.broadcasted_iota(jnp.int32, (RCH // SCH, RCH), 0)
          ).astype(jnp.float32)

    for c in range(T // RCH):
        rows = pl.ds(c * RCH, RCH)
        xb = x_ref[rows, :]
        logits = lax.dot_general(xb, wgate, (((1,), (1,)), ((), ())),
                                 preferred_element_type=jnp.float32)
        m = jnp.max(logits, axis=1, keepdims=True)
        ex = jnp.exp(logits - m)
        probs = ex / jnp.sum(ex, axis=1, keepdims=True)

        eidx = lax.broadcasted_iota(jnp.int32, probs.shape, 1)
        p0 = jnp.max(probs, axis=1, keepdims=True)
        e0 = jnp.min(jnp.where(probs == p0, eidx, NUM_EXPERTS),
                     axis=1, keepdims=True)
        oh0 = (eidx == e0).astype(jnp.float32)
        masked = jnp.where(eidx == e0, -jnp.inf, probs)
        p1 = jnp.max(masked, axis=1, keepdims=True)
        e1 = jnp.min(jnp.where(masked == p1, eidx, NUM_EXPERTS),
                     axis=1, keepdims=True)
        oh1 = (eidx == e1).astype(jnp.float32)

        denom = p0 + p1
        w0_ref[rows, :] = p0 / denom
        w1_ref[rows, :] = p1 / denom
        e0_ref[rows, :] = e0
        e1_ref[rows, :] = e1

        h = oh0 + oh1
        h_ref[rows, :] = h
        s_ref[pl.ds(c * (RCH // SCH), RCH // SCH), :] = lax.dot_general(
            g8, h, (((1,), (0,)), ((), ())),
            preferred_element_type=jnp.float32)
        psum = psum + jnp.sum(probs * h, axis=0, keepdims=True)

    s = s_ref[...]
    nsc = T // SCH
    slt32 = (lax.broadcasted_iota(jnp.int32, (nsc, nsc), 0)
             > lax.broadcasted_iota(jnp.int32, (nsc, nsc), 1)
             ).astype(jnp.float32)
    p32 = lax.dot_general(slt32, s, (((1,), (0,)), ((), ())),
                          preferred_element_type=jnp.float32)
    counts = jnp.sum(s, axis=0, keepdims=True)
    cnt_i = counts.astype(jnp.int32)
    pc = ((cnt_i + (BM - 1)) // BM) * BM
    sut = (lax.broadcasted_iota(jnp.int32, (NUM_EXPERTS, NUM_EXPERTS), 0)
           < lax.broadcasted_iota(jnp.int32, (NUM_EXPERTS, NUM_EXPERTS), 1)
           ).astype(jnp.float32)
    startsf = lax.dot_general(pc.astype(jnp.float32), sut,
                              (((1,), (0,)), ((), ())),
                              preferred_element_type=jnp.float32)
    starts = startsf.astype(jnp.int32)
    nact = jnp.sum(pc) // BM

    bs = lax.broadcasted_iota(jnp.int32, (NB1, NUM_EXPERTS), 0) * BM
    eix = lax.broadcasted_iota(jnp.int32, (NB1, NUM_EXPERTS), 1)
    hit = jnp.logical_and(bs >= starts, bs < starts + pc).astype(jnp.int32)
    elast = jnp.max(jnp.where(counts > 0, eix[:1, :], -1))
    rowi = lax.broadcasted_iota(jnp.int32, (NB1, 1), 0)
    bexp = jnp.sum(eix * hit, axis=1, keepdims=True)
    bexp = jnp.where(rowi < nact, bexp, elast)
    bexp = jnp.where(rowi == NB, nact, bexp)
    bexp_ref[...] = bexp

    slt = (lax.broadcasted_iota(jnp.int32, (SCH, SCH), 0)
           > lax.broadcasted_iota(jnp.int32, (SCH, SCH), 1)
           ).astype(jnp.float32)
    for c in range(nsc):
        rows = pl.ds(c * SCH, SCH)
        intra = lax.dot_general(slt, h_ref[rows, :], (((1,), (0,)), ((), ())),
                                preferred_element_type=jnp.float32)
        cums = intra + lax.slice(p32, (c, 0), (c + 1, NUM_EXPERTS)) + startsf
        eidx = lax.broadcasted_iota(jnp.int32, (SCH, NUM_EXPERTS), 1)
        oh0 = (eidx == e0_ref[rows, :]).astype(jnp.float32)
        oh1 = (eidx == e1_ref[rows, :]).astype(jnp.float32)
        pos0_ref[rows, :] = jnp.sum(cums * oh0, axis=1,
                                    keepdims=True).astype(jnp.int32)
        pos1_ref[rows, :] = jnp.sum(cums * oh1, axis=1,
                                    keepdims=True).astype(jnp.int32)

    p_expert = psum / jnp.float32(T)
    p_tok = counts / jnp.float32(ASSIGN)
    aux_ref[0, 0] = jnp.sum(p_expert * p_tok) * NUM_EXPERTS * AUX_COEF


def _router(x2d, W_gate, interpret=False):
    return pl.pallas_call(
        _router_kernel,
        in_specs=[
            pl.BlockSpec((T, HIDDEN), lambda: (0, 0)),
            pl.BlockSpec((NUM_EXPERTS, HIDDEN), lambda: (0, 0)),
        ],
        out_specs=[
            pl.BlockSpec((T, 1), lambda: (0, 0)),
            pl.BlockSpec((T, 1), lambda: (0, 0)),
            pl.BlockSpec((T, 1), lambda: (0, 0)),
            pl.BlockSpec((T, 1), lambda: (0, 0)),
            pl.BlockSpec((NB1, 1), lambda: (0, 0)),
            pl.BlockSpec(memory_space=pltpu.SMEM),
        ],
        out_shape=[
            jax.ShapeDtypeStruct((T, 1), jnp.int32),
            jax.ShapeDtypeStruct((T, 1), jnp.int32),
            jax.ShapeDtypeStruct((T, 1), jnp.float32),
            jax.ShapeDtypeStruct((T, 1), jnp.float32),
            jax.ShapeDtypeStruct((NB1, 1), jnp.int32),
            jax.ShapeDtypeStruct((1, 1), jnp.float32),
        ],
        scratch_shapes=[
            pltpu.VMEM((T, 1), jnp.int32),
            pltpu.VMEM((T, 1), jnp.int32),
            pltpu.VMEM((T, NUM_EXPERTS), jnp.float32),
            pltpu.VMEM((T // SCH, NUM_EXPERTS), jnp.float32),
        ],
        interpret=interpret,
    )(x2d, W_gate)


def _sc_mesh():
    return plsc.VectorSubcoreMesh(core_axis_name="c", subcore_axis_name="s",
                                  num_cores=2, num_subcores=16)


def _dispatch_body(x_hbm, pos0_hbm, pos1_hbm, xs_hbm,
                   idx0_v, idx1_v, rows_v, sem0, sem1):
    wid = lax.axis_index("s") * 2 + lax.axis_index("c")
    for sub in range(TPW // DCHUNK):
        base = wid * TPW + sub * DCHUNK
        pltpu.sync_copy(pos0_hbm.at[pl.ds(base, DCHUNK)], idx0_v)
        pltpu.sync_copy(pos1_hbm.at[pl.ds(base, DCHUNK)], idx1_v)
        pltpu.sync_copy(x_hbm.at[pl.ds(base, DCHUNK)], rows_v)
        d0 = pltpu.async_copy(rows_v, xs_hbm.at[idx0_v], sem0)
        d1 = pltpu.async_copy(rows_v, xs_hbm.at[idx1_v], sem1)
        d0.wait()
        d1.wait()


@functools.lru_cache(maxsize=None)
def _make_dispatch():
    return pl.kernel(
        _dispatch_body,
        out_type=jax.ShapeDtypeStruct((ROWS, HIDDEN), jnp.float32),
        mesh=_sc_mesh(),
        scratch_types=[
            pltpu.VMEM((DCHUNK,), jnp.int32),
            pltpu.VMEM((DCHUNK,), jnp.int32),
            pltpu.VMEM((DCHUNK, HIDDEN), jnp.float32),
            pltpu.SemaphoreType.DMA,
            pltpu.SemaphoreType.DMA,
        ],
    )


def _ffn_kernel(bexp_ref, xs_ref, wg_ref, wu_ref, wd_ref, ys_ref):
    b = pl.program_id(0)
    i = pl.program_id(1)
    nblk = bexp_ref[NB]

    @pl.when(b < nblk)
    def _active():
        @pl.when(i == 0)
        def _init():
            ys_ref[...] = jnp.zeros_like(ys_ref)

        xb = xs_ref[...]
        g = lax.dot_general(xb, wg_ref[0], (((1,), (1,)), ((), ())),
                            preferred_element_type=jnp.float32)
        u = lax.dot_general(xb, wu_ref[0], (((1,), (1,)), ((), ())),
                            preferred_element_type=jnp.float32)
        h = (g * jax.nn.sigmoid(g)) * u
        ys_ref[...] += lax.dot_general(h, wd_ref[0], (((1,), (1,)), ((), ())),
                                       preferred_element_type=jnp.float32)


def _row_clamp(b, be):
    return jnp.minimum(b, be[NB] - 1)


def _i_clamp(b, i, be):
    return jnp.where(b < be[NB], i, NI - 1)


def _ffn(bexp, xs, Wg, Wu, Wd, interpret=False):
    grid_spec = pltpu.PrefetchScalarGridSpec(
        num_scalar_prefetch=1,
        grid=(NB, NI),
        in_specs=[
            pl.BlockSpec((BM, HIDDEN), lambda b, i, be: (_row_clamp(b, be), 0)),
            pl.BlockSpec((1, INT_BLK, HIDDEN),
                         lambda b, i, be: (be[b], _i_clamp(b, i, be), 0)),
            pl.BlockSpec((1, INT_BLK, HIDDEN),
                         lambda b, i, be: (be[b], _i_clamp(b, i, be), 0)),
            pl.BlockSpec((1, HIDDEN, INT_BLK),
                         lambda b, i, be: (be[b], 0, _i_clamp(b, i, be))),
        ],
        out_specs=pl.BlockSpec((BM, HIDDEN),
                               lambda b, i, be: (_row_clamp(b, be), 0)),
    )
    return pl.pallas_call(
        _ffn_kernel,
        grid_spec=grid_spec,
        out_shape=jax.ShapeDtypeStruct((ROWS, HIDDEN), jnp.float32),
        interpret=interpret,
    )(bexp, xs, Wg, Wu, Wd)


def _combine_body(ys_hbm, pos0_hbm, pos1_hbm, w0_hbm, w1_hbm, out_hbm,
                  idx0_v, idx1_v, w0_v, w1_v, a_v, b_v, sem0, sem1):
    wid = lax.axis_index("s") * 2 + lax.axis_index("c")
    for sub in range(TPW // CCHUNK):
        base = wid * TPW + sub * CCHUNK
        pltpu.sync_copy(pos0_hbm.at[pl.ds(base, CCHUNK)], idx0_v)
        pltpu.sync_copy(pos1_hbm.at[pl.ds(base, CCHUNK)], idx1_v)
        pltpu.sync_copy(w0_hbm.at[pl.ds(base, CCHUNK)], w0_v)
        pltpu.sync_copy(w1_hbm.at[pl.ds(base, CCHUNK)], w1_v)
        d0 = pltpu.async_copy(ys_hbm.at[idx0_v], a_v, sem0)
        d1 = pltpu.async_copy(ys_hbm.at[idx1_v], b_v, sem1)
        d0.wait()
        d1.wait()

        wvecs = [(w0_v[pl.ds(g * 16, 16)], w1_v[pl.ds(g * 16, 16)])
                 for g in range(CCHUNK // 16)]
        for i in range(CCHUNK):
            w0s = wvecs[i // 16][0][i % 16]
            w1s = wvecs[i // 16][1][i % 16]

            def col_body(j, _, i=i, w0s=w0s, w1s=w1s):
                for k in range(8):
                    sl = pl.ds(j * 128 + k * 16, 16)
                    a_v[i, sl] = a_v[i, sl] * w0s + b_v[i, sl] * w1s
                return 0

            lax.fori_loop(0, HIDDEN // 128, col_body, 0)
        pltpu.sync_copy(a_v, out_hbm.at[pl.ds(base, CCHUNK)])


@functools.lru_cache(maxsize=None)
def _make_combine():
    return pl.kernel(
        _combine_body,
        out_type=jax.ShapeDtypeStruct((T, HIDDEN), jnp.float32),
        mesh=_sc_mesh(),
        scratch_types=[
            pltpu.VMEM((CCHUNK,), jnp.int32),
            pltpu.VMEM((CCHUNK,), jnp.int32),
            pltpu.VMEM((CCHUNK,), jnp.float32),
            pltpu.VMEM((CCHUNK,), jnp.float32),
            pltpu.VMEM((CCHUNK, HIDDEN), jnp.float32),
            pltpu.VMEM((CCHUNK, HIDDEN), jnp.float32),
            pltpu.SemaphoreType.DMA,
            pltpu.SemaphoreType.DMA,
        ],
    )


@jax.jit
def _moe(x2d, W_gate, Wg, Wu, Wd):
    pos0, pos1, w0, w1, bexp, aux = _router(x2d, W_gate)
    p0 = pos0.reshape(T)
    p1 = pos1.reshape(T)
    xs = _make_dispatch()(x2d, p0, p1)
    ys = _ffn(bexp.reshape(NB1), xs, Wg, Wu, Wd)
    y2d = _make_combine()(ys, p0, p1, w0.reshape(T), w1.reshape(T))
    return y2d, aux[0, 0]


def kernel(x, W_gate, Wg, Wu, Wd):
    bsz, seq, hid = x.shape
    x2d = x.reshape(-1, hid)
    y, aux = _moe(x2d, W_gate, Wg, Wu, Wd)
    return y.reshape(bsz, seq, hid), aux


# speedup vs baseline: 2.3983x; 1.1270x over previous
import functools

import jax
import jax.numpy as jnp
from jax import lax
from jax.experimental import pallas as pl
from jax.experimental.pallas import tpu as pltpu
from jax.experimental.pallas import tpu_sc as plsc

HIDDEN = 1024
INTER = 2048
NUM_EXPERTS = 8
TOP_K = 2
AUX_COEF = 0.001
T = 4096
ASSIGN = T * TOP_K

BM = 512
ROWS = ASSIGN + NUM_EXPERTS * BM
NB = ROWS // BM
NB1 = NB + 1
INT_BLK = 1024
NI = INTER // INT_BLK

RCH = 1024
SCH = 128

NW = 32
TPW = T // NW
DCHUNK = 64
CCHUNK = 32


def _router_kernel(x_ref, wgate_ref,
                   pos0_ref, pos1_ref, w0_ref, w1_ref, bexp_ref, aux_ref,
                   e0_ref, e1_ref, h_ref, s_ref):
    wgate = wgate_ref[...]
    psum = jnp.zeros((1, NUM_EXPERTS), jnp.float32)

    g8 = (lax.broadcasted_iota(jnp.int32, (RCH // SCH, RCH), 1) // SCH
          == lax.broadcasted_iota(jnp.int32, (RCH // SCH, RCH), 0)
          ).astype(jnp.float32)

    for c in range(T // RCH):
        rows = pl.ds(c * RCH, RCH)
        xb = x_ref[rows, :]
        logits = lax.dot_general(xb, wgate, (((1,), (1,)), ((), ())),
                                 preferred_element_type=jnp.float32)
        m = jnp.max(logits, axis=1, keepdims=True)
        ex = jnp.exp(logits - m)
        probs = ex / jnp.sum(ex, axis=1, keepdims=True)

        eidx = lax.broadcasted_iota(jnp.int32, probs.shape, 1)
        p0 = jnp.max(probs, axis=1, keepdims=True)
        e0 = jnp.min(jnp.where(probs == p0, eidx, NUM_EXPERTS),
                     axis=1, keepdims=True)
        oh0 = (eidx == e0).astype(jnp.float32)
        masked = jnp.where(eidx == e0, -jnp.inf, probs)
        p1 = jnp.max(masked, axis=1, keepdims=True)
        e1 = jnp.min(jnp.where(masked == p1, eidx, NUM_EXPERTS),
                     axis=1, keepdims=True)
        oh1 = (eidx == e1).astype(jnp.float32)

        denom = p0 + p1
        w0_ref[rows, :] = p0 / denom
        w1_ref[rows, :] = p1 / denom
        e0_ref[rows, :] = e0
        e1_ref[rows, :] = e1

        h = oh0 + oh1
        h_ref[rows, :] = h
        s_ref[pl.ds(c * (RCH // SCH), RCH // SCH), :] = lax.dot_general(
            g8, h, (((1,), (0,)), ((), ())),
            preferred_element_type=jnp.float32)
        psum = psum + jnp.sum(probs * h, axis=0, keepdims=True)

    s = s_ref[...]
    nsc = T // SCH
    slt32 = (lax.broadcasted_iota(jnp.int32, (nsc, nsc), 0)
             > lax.broadcasted_iota(jnp.int32, (nsc, nsc), 1)
             ).astype(jnp.float32)
    p32 = lax.dot_general(slt32, s, (((1,), (0,)), ((), ())),
                          preferred_element_type=jnp.float32)
    counts = jnp.sum(s, axis=0, keepdims=True)
    cnt_i = counts.astype(jnp.int32)
    pc = ((cnt_i + (BM - 1)) // BM) * BM
    sut = (lax.broadcasted_iota(jnp.int32, (NUM_EXPERTS, NUM_EXPERTS), 0)
           < lax.broadcasted_iota(jnp.int32, (NUM_EXPERTS, NUM_EXPERTS), 1)
           ).astype(jnp.float32)
    startsf = lax.dot_general(pc.astype(jnp.float32), sut,
                              (((1,), (0,)), ((), ())),
                              preferred_element_type=jnp.float32)
    starts = startsf.astype(jnp.int32)
    nact = jnp.sum(pc) // BM

    bs = lax.broadcasted_iota(jnp.int32, (NB1, NUM_EXPERTS), 0) * BM
    eix = lax.broadcasted_iota(jnp.int32, (NB1, NUM_EXPERTS), 1)
    hit = jnp.logical_and(bs >= starts, bs < starts + pc).astype(jnp.int32)
    elast = jnp.max(jnp.where(counts > 0, eix[:1, :], -1))
    rowi = lax.broadcasted_iota(jnp.int32, (NB1, 1), 0)
    bexp = jnp.sum(eix * hit, axis=1, keepdims=True)
    bexp = jnp.where(rowi < nact, bexp, elast)
    bexp = jnp.where(rowi == NB, nact, bexp)
    bexp_ref[...] = bexp

    slt = (lax.broadcasted_iota(jnp.int32, (SCH, SCH), 0)
           > lax.broadcasted_iota(jnp.int32, (SCH, SCH), 1)
           ).astype(jnp.float32)
    for c in range(nsc):
        rows = pl.ds(c * SCH, SCH)
        intra = lax.dot_general(slt, h_ref[rows, :], (((1,), (0,)), ((), ())),
                                preferred_element_type=jnp.float32)
        cums = intra + lax.slice(p32, (c, 0), (c + 1, NUM_EXPERTS)) + startsf
        eidx = lax.broadcasted_iota(jnp.int32, (SCH, NUM_EXPERTS), 1)
        oh0 = (eidx == e0_ref[rows, :]).astype(jnp.float32)
        oh1 = (eidx == e1_ref[rows, :]).astype(jnp.float32)
        pos0_ref[rows, :] = jnp.sum(cums * oh0, axis=1,
                                    keepdims=True).astype(jnp.int32)
        pos1_ref[rows, :] = jnp.sum(cums * oh1, axis=1,
                                    keepdims=True).astype(jnp.int32)

    p_expert = psum / jnp.float32(T)
    p_tok = counts / jnp.float32(ASSIGN)
    aux_ref[0, 0] = jnp.sum(p_expert * p_tok) * NUM_EXPERTS * AUX_COEF


def _router(x2d, W_gate, interpret=False):
    return pl.pallas_call(
        _router_kernel,
        in_specs=[
            pl.BlockSpec((T, HIDDEN), lambda: (0, 0)),
            pl.BlockSpec((NUM_EXPERTS, HIDDEN), lambda: (0, 0)),
        ],
        out_specs=[
            pl.BlockSpec((T, 1), lambda: (0, 0)),
            pl.BlockSpec((T, 1), lambda: (0, 0)),
            pl.BlockSpec((T, 1), lambda: (0, 0)),
            pl.BlockSpec((T, 1), lambda: (0, 0)),
            pl.BlockSpec((NB1, 1), lambda: (0, 0)),
            pl.BlockSpec(memory_space=pltpu.SMEM),
        ],
        out_shape=[
            jax.ShapeDtypeStruct((T, 1), jnp.int32),
            jax.ShapeDtypeStruct((T, 1), jnp.int32),
            jax.ShapeDtypeStruct((T, 1), jnp.float32),
            jax.ShapeDtypeStruct((T, 1), jnp.float32),
            jax.ShapeDtypeStruct((NB1, 1), jnp.int32),
            jax.ShapeDtypeStruct((1, 1), jnp.float32),
        ],
        scratch_shapes=[
            pltpu.VMEM((T, 1), jnp.int32),
            pltpu.VMEM((T, 1), jnp.int32),
            pltpu.VMEM((T, NUM_EXPERTS), jnp.float32),
            pltpu.VMEM((T // SCH, NUM_EXPERTS), jnp.float32),
        ],
        interpret=interpret,
    )(x2d, W_gate)


def _sc_mesh():
    return plsc.VectorSubcoreMesh(core_axis_name="c", subcore_axis_name="s",
                                  num_cores=2, num_subcores=16)


def _dispatch_body(x_hbm, pos0_hbm, pos1_hbm, xs_hbm,
                   idx0_v, idx1_v, rows_v, sem0, sem1):
    wid = lax.axis_index("s") * 2 + lax.axis_index("c")
    for sub in range(TPW // DCHUNK):
        base = wid * TPW + sub * DCHUNK
        pltpu.sync_copy(pos0_hbm.at[pl.ds(base, DCHUNK)], idx0_v)
        pltpu.sync_copy(pos1_hbm.at[pl.ds(base, DCHUNK)], idx1_v)
        pltpu.sync_copy(x_hbm.at[pl.ds(base, DCHUNK)], rows_v)
        d0 = pltpu.async_copy(rows_v, xs_hbm.at[idx0_v], sem0)
        d1 = pltpu.async_copy(rows_v, xs_hbm.at[idx1_v], sem1)
        d0.wait()
        d1.wait()


@functools.lru_cache(maxsize=None)
def _make_dispatch():
    return pl.kernel(
        _dispatch_body,
        out_type=jax.ShapeDtypeStruct((ROWS, HIDDEN), jnp.float32),
        mesh=_sc_mesh(),
        scratch_types=[
            pltpu.VMEM((DCHUNK,), jnp.int32),
            pltpu.VMEM((DCHUNK,), jnp.int32),
            pltpu.VMEM((DCHUNK, HIDDEN), jnp.float32),
            pltpu.SemaphoreType.DMA,
            pltpu.SemaphoreType.DMA,
        ],
    )


def _ffn_kernel(bexp_ref, xs_ref, wg_ref, wu_ref, wd_ref, ys_ref):
    b = pl.program_id(0)
    i = pl.program_id(1)
    nblk = bexp_ref[NB]

    @pl.when(b < nblk)
    def _active():
        @pl.when(i == 0)
        def _init():
            ys_ref[...] = jnp.zeros_like(ys_ref)

        xb = xs_ref[...]
        g = lax.dot_general(xb, wg_ref[0], (((1,), (1,)), ((), ())),
                            preferred_element_type=jnp.float32)
        u = lax.dot_general(xb, wu_ref[0], (((1,), (1,)), ((), ())),
                            preferred_element_type=jnp.float32)
        h = (g * jax.nn.sigmoid(g)) * u
        ys_ref[...] += lax.dot_general(h, wd_ref[0], (((1,), (1,)), ((), ())),
                                       preferred_element_type=jnp.float32)


def _row_clamp(b, be):
    return jnp.minimum(b, be[NB] - 1)


def _i_clamp(b, i, be):
    nblk = be[NB]
    i_act = jnp.where(b % 2 == 1, NI - 1 - i, i)
    i_dead = jnp.where((nblk - 1) % 2 == 1, 0, NI - 1)
    return jnp.where(b < nblk, i_act, i_dead)


def _ffn(bexp, xs, Wg, Wu, Wd, interpret=False):
    grid_spec = pltpu.PrefetchScalarGridSpec(
        num_scalar_prefetch=1,
        grid=(NB, NI),
        in_specs=[
            pl.BlockSpec((BM, HIDDEN), lambda b, i, be: (_row_clamp(b, be), 0)),
            pl.BlockSpec((1, INT_BLK, HIDDEN),
                         lambda b, i, be: (be[b], _i_clamp(b, i, be), 0)),
            pl.BlockSpec((1, INT_BLK, HIDDEN),
                         lambda b, i, be: (be[b], _i_clamp(b, i, be), 0)),
            pl.BlockSpec((1, HIDDEN, INT_BLK),
                         lambda b, i, be: (be[b], 0, _i_clamp(b, i, be))),
        ],
        out_specs=pl.BlockSpec((BM, HIDDEN),
                               lambda b, i, be: (_row_clamp(b, be), 0)),
    )
    return pl.pallas_call(
        _ffn_kernel,
        grid_spec=grid_spec,
        out_shape=jax.ShapeDtypeStruct((ROWS, HIDDEN), jnp.float32),
        interpret=interpret,
    )(bexp, xs, Wg, Wu, Wd)


def _combine_body(ys_hbm, pos0_hbm, pos1_hbm, w0_hbm, w1_hbm, out_hbm,
                  idx0_v, idx1_v, w0_v, w1_v, a_v, b_v, sem0, sem1):
    wid = lax.axis_index("s") * 2 + lax.axis_index("c")
    for sub in range(TPW // CCHUNK):
        base = wid * TPW + sub * CCHUNK
        pltpu.sync_copy(pos0_hbm.at[pl.ds(base, CCHUNK)], idx0_v)
        pltpu.sync_copy(pos1_hbm.at[pl.ds(base, CCHUNK)], idx1_v)
        pltpu.sync_copy(w0_hbm.at[pl.ds(base, CCHUNK)], w0_v)
        pltpu.sync_copy(w1_hbm.at[pl.ds(base, CCHUNK)], w1_v)
        d0 = pltpu.async_copy(ys_hbm.at[idx0_v], a_v, sem0)
        d1 = pltpu.async_copy(ys_hbm.at[idx1_v], b_v, sem1)
        d0.wait()
        d1.wait()

        wvecs = [(w0_v[pl.ds(g * 16, 16)], w1_v[pl.ds(g * 16, 16)])
                 for g in range(CCHUNK // 16)]
        for i in range(CCHUNK):
            w0s = wvecs[i // 16][0][i % 16]
            w1s = wvecs[i // 16][1][i % 16]

            def col_body(j, _, i=i, w0s=w0s, w1s=w1s):
                for k in range(8):
                    sl = pl.ds(j * 128 + k * 16, 16)
                    a_v[i, sl] = a_v[i, sl] * w0s + b_v[i, sl] * w1s
                return 0

            lax.fori_loop(0, HIDDEN // 128, col_body, 0)
        pltpu.sync_copy(a_v, out_hbm.at[pl.ds(base, CCHUNK)])


@functools.lru_cache(maxsize=None)
def _make_combine():
    return pl.kernel(
        _combine_body,
        out_type=jax.ShapeDtypeStruct((T, HIDDEN), jnp.float32),
        mesh=_sc_mesh(),
        scratch_types=[
            pltpu.VMEM((CCHUNK,), jnp.int32),
            pltpu.VMEM((CCHUNK,), jnp.int32),
            pltpu.VMEM((CCHUNK,), jnp.float32),
            pltpu.VMEM((CCHUNK,), jnp.float32),
            pltpu.VMEM((CCHUNK, HIDDEN), jnp.float32),
            pltpu.VMEM((CCHUNK, HIDDEN), jnp.float32),
            pltpu.SemaphoreType.DMA,
            pltpu.SemaphoreType.DMA,
        ],
    )


@jax.jit
def _moe(x2d, W_gate, Wg, Wu, Wd):
    pos0, pos1, w0, w1, bexp, aux = _router(x2d, W_gate)
    p0 = pos0.reshape(T)
    p1 = pos1.reshape(T)
    xs = _make_dispatch()(x2d, p0, p1)
    ys = _ffn(bexp.reshape(NB1), xs, Wg, Wu, Wd)
    y2d = _make_combine()(ys, p0, p1, w0.reshape(T), w1.reshape(T))
    return y2d, aux[0, 0]


def kernel(x, W_gate, Wg, Wu, Wd):
    bsz, seq, hid = x.shape
    x2d = x.reshape(-1, hid)
    y, aux = _moe(x2d, W_gate, Wg, Wu, Wd)
    return y.reshape(bsz, seq, hid), aux
